# Initial kernel scaffold; baseline (speedup 1.0000x reference)
import jax
import jax.numpy as jnp
from jax.experimental import pallas as pl


def kernel(xyz, points, affine_alpha, affine_beta):
    raise NotImplementedError("write your pallas kernel here")


# speedup vs baseline: 21.9781x; 21.9781x over previous
import functools

import numpy as np
import jax
import jax.numpy as jnp
from jax import lax
from jax.experimental import pallas as pl
from jax.experimental.pallas import tpu as pltpu
from jax.experimental.pallas import tpu_sc as plsc

_REDUCE = 4
_K = 32
_R2 = np.float32(0.2 * 0.2)
_QT = 128


def _fps_body(xs_ref, ys_ref, zs_ref, oi_ref, ox_ref, oy_ref, oz_ref):
    xs = xs_ref[...]
    ys = ys_ref[...]
    zs = zs_ref[...]
    R2, Nh = xs.shape
    S = oi_ref.shape[1]
    row_i = lax.broadcasted_iota(jnp.int32, (R2, 1), 0)
    parity_f = (row_i & 1).astype(jnp.float32)
    n_map = (parity_f * Nh
             + lax.broadcasted_iota(jnp.int32, (R2, Nh), 1).astype(jnp.float32))
    s_iota = lax.broadcasted_iota(jnp.int32, (1, S), 1)
    big = jnp.float32(2 * Nh)
    even = (row_i & 1) == 0

    def partner(v):
        up = pltpu.roll(v, 1, 0)
        dn = pltpu.roll(v, R2 - 1, 0)
        return jnp.where(even, dn, up)

    def pair(v, op):
        return op(v, partner(v))

    n_i = (row_i & 1) * Nh + lax.broadcasted_iota(jnp.int32, (R2, Nh), 1)
    imax = jnp.int32(2147483647)

    def mkkeys(v):
        vb = lax.bitcast_convert_type(v, jnp.int32)
        ka = (n_i << 19) | lax.shift_right_logical(vb, 13)
        kb = (n_i << 13) | (vb & 0x1FFF)
        return ka, kb

    kxa, kxb = mkkeys(xs)
    kya, kyb = mkkeys(ys)
    kza, kzb = mkkeys(zs)

    def unpack(ka, kb):
        bits = ((ka & 0x7FFFF) << 13) | (kb & 0x1FFF)
        return lax.bitcast_convert_type(bits, jnp.float32)

    def body(i, st):
        dist, far, cx, cy, cz, acc_i, acc_x, acc_y, acc_z = st
        here = s_iota == i
        acc_i = jnp.where(here, far, acc_i)
        acc_x = jnp.where(here, cx, acc_x)
        acc_y = jnp.where(here, cy, acc_y)
        acc_z = jnp.where(here, cz, acc_z)
        dx = xs - cx
        dy = ys - cy
        dz = zs - cz
        dn = dx * dx + dy * dy + dz * dz
        dist = jnp.minimum(dist, dn)
        m = pair(jnp.max(dist, axis=1, keepdims=True), jnp.maximum)
        sel = dist == m

        def mw(k):
            return pair(jnp.min(jnp.where(sel, k, imax),
                                axis=1, keepdims=True), jnp.minimum)

        xa, xb2 = mw(kxa), mw(kxb)
        ya, yb2 = mw(kya), mw(kyb)
        za, zb2 = mw(kza), mw(kzb)
        far2 = lax.shift_right_logical(xa, 19)
        return (dist, far2, unpack(xa, xb2), unpack(ya, yb2),
                unpack(za, zb2), acc_i, acc_x, acc_y, acc_z)

    dist0 = jnp.full((R2, Nh), 1e10, dtype=jnp.float32)
    zi = lax.broadcasted_iota(jnp.int32, (R2, S), 1) + row_i
    zf = zi.astype(jnp.float32)

    def col0(v):
        return jnp.where(even, v[:, 0:1], pltpu.roll(v[:, 0:1], 1, 0))

    def body2(i2, st):
        return body(2 * i2 + 1, body(2 * i2, st))

    st0 = (dist0, jnp.zeros((R2, 1), jnp.int32),
           col0(xs), col0(ys), col0(zs), zi, zf, zf, zf)
    _, _, _, _, _, acc_i, acc_x, acc_y, acc_z = lax.fori_loop(
        0, S // 2, body2, st0)
    oi_ref[...] = acc_i
    ox_ref[...] = acc_x
    oy_ref[...] = acc_y
    oz_ref[...] = acc_z


def _run_fps(xyz, S):
    B, N, _ = xyz.shape
    coords = [xyz[:, :, c].reshape(2 * B, N // 2) for c in range(3)]
    oi, ox, oy, oz = pl.pallas_call(
        _fps_body,
        out_shape=[jax.ShapeDtypeStruct((2 * B, S), jnp.int32)]
        + [jax.ShapeDtypeStruct((2 * B, S), jnp.float32)] * 3,
    )(*coords)
    fps_idx = oi.reshape(B, 2, S)[:, 0, :]
    new_xyz_t = jnp.stack(
        [v.reshape(B, 2, S)[:, 0, :] for v in (ox, oy, oz)], axis=1)
    return fps_idx, new_xyz_t


def _ballq_body(q_ref, xyz_ref, out_ref):
    b = pl.program_id(0)
    X = xyz_ref[0]
    xr = X[:, 0:1]
    yr = X[:, 1:2]
    zr = X[:, 2:3]
    Q = q_ref[0]
    qx = Q[0:1, :]
    qy = Q[1:2, :]
    qz = Q[2:3, :]
    dx = qx - xr
    dy = qy - yr
    dz = qz - zr
    d = dx * dx + dy * dy + dz * dz
    n_pts, qt = d.shape
    big = jnp.int32(n_pts)
    nw = 128
    nbit = n_pts // nw
    mask = (d <= _R2).astype(jnp.int32)
    W = mask[0:nw, :]
    for j in range(1, nbit):
        W = W | (mask[j * nw:(j + 1) * nw, :] << j)
    w_iota = lax.broadcasted_iota(jnp.int32, (nw, qt), 0)
    cols = []
    for _ in range(_K):
        lsb = W & (-W)
        ebits = lax.shift_right_logical(
            lax.bitcast_convert_type(lsb.astype(jnp.float32), jnp.int32), 23)
        j = (ebits & 0xFF) - 127
        n_cand = jnp.where(W != 0, (j << 7) + w_iota, big)
        m = jnp.min(n_cand, axis=0, keepdims=True)
        cols.append(m)
        bit = jnp.int32(1) << (m >> 7)
        W = jnp.where(w_iota == (m & 127), W & ~bit, W)
    idx = jnp.concatenate(cols, axis=0)
    idx = jnp.where(idx >= big, cols[0], idx)
    out_ref[0, 0] = idx + b * n_pts


def _run_ballq(new_xyz_t, xyz):
    B, _, S = new_xyz_t.shape
    N = xyz.shape[1]
    return pl.pallas_call(
        _ballq_body,
        grid=(B, S // _QT),
        in_specs=[
            pl.BlockSpec((1, 3, _QT), lambda b, t: (b, 0, t)),
            pl.BlockSpec((1, N, 3), lambda b, t: (b, 0, 0)),
        ],
        out_specs=pl.BlockSpec((1, 1, _K, _QT), lambda b, t: (b, t, 0, 0)),
        out_shape=jax.ShapeDtypeStruct((B, S // _QT, _K, _QT), jnp.int32),
    )(new_xyz_t, xyz)


def _sc_mesh():
    return plsc.VectorSubcoreMesh(core_axis_name="c", subcore_axis_name="s")


def _sc_info():
    try:
        info = plsc.get_sparse_core_info()
        return info.num_cores, info.num_subcores
    except Exception:
        return 2, 16


def _run_c2(pts_flat, idx_all, gidx, alpha, beta):
    C = pts_flat.shape[1]
    SQ = gidx.shape[0]
    nc, ns = _sc_info()
    nper = SQ // (nc * ns)
    cq = 4
    nchunks = nper // cq

    @functools.partial(
        pl.kernel,
        out_type=jax.ShapeDtypeStruct((SQ, C), jnp.float32),
        mesh=_sc_mesh(),
        scratch_types=[pltpu.VMEM((cq * _K,), jnp.int32),
                       pltpu.VMEM((cq * _K,), jnp.int32),
                       pltpu.VMEM((cq * _K, C), jnp.float32),
                       pltpu.VMEM((cq * _K, C), jnp.float32),
                       pltpu.VMEM((nper,), jnp.int32),
                       pltpu.VMEM((nper, C), jnp.float32),
                       pltpu.VMEM((cq, C), jnp.float32),
                       pltpu.VMEM((C,), jnp.float32),
                       pltpu.VMEM((C,), jnp.float32),
                       pltpu.SemaphoreType.DMA,
                       pltpu.SemaphoreType.DMA,
                       pltpu.SemaphoreType.DMA],
    )
    def c2(pts, idxh, gidxh, alphah, betah, out,
           idx0, idx1, rows0, rows1, fidx_v, anch_all, out_v, al_v, be_v,
           s0, s1, sa):
        idx_v = (idx0, idx1)
        rows_v = (rows0, rows1)
        sem = (s0, s1)
        wid = lax.axis_index("s") * nc + lax.axis_index("c")
        qbase = wid * nper
        pltpu.sync_copy(alphah, al_v)
        pltpu.sync_copy(betah, be_v)

        def issue(ci, sl):
            q0 = qbase + ci * cq
            pltpu.sync_copy(idxh.at[pl.ds(q0 * _K, cq * _K)], idx_v[sl])
            pltpu.async_copy(pts.at[idx_v[sl]], rows_v[sl], sem[sl])

        issue(0, 0)
        pltpu.sync_copy(gidxh.at[pl.ds(qbase, nper)], fidx_v)
        pltpu.async_copy(pts.at[fidx_v], anch_all, sa).wait()

        def wait_compute(ci, sl):
            pltpu.make_async_copy(pts.at[idx_v[sl]], rows_v[sl],
                                  sem[sl]).wait()

            def q_body(qi, _):
                for c in range(C // 16):
                    slc = pl.ds(c * 16, 16)
                    av = anch_all[ci * cq + qi, slc]
                    alv = al_v[slc]
                    bev = be_v[slc]
                    acc = jnp.full((16,), -jnp.inf, jnp.float32)
                    for k in range(_K):
                        v = rows_v[sl][qi * _K + k, slc]
                        acc = jnp.maximum(acc, (v - av) * alv + bev)
                    out_v[qi, slc] = acc
                return 0

            lax.fori_loop(0, cq, q_body, 0)
            pltpu.sync_copy(out_v, out.at[pl.ds(qbase + ci * cq, cq)])

        def pair_body(cp, _):
            ci = cp * 2
            issue(ci + 1, 1)
            wait_compute(ci, 0)

            @pl.when(cp + 1 < nchunks // 2)
            def _prefetch():
                issue(ci + 2, 0)

            wait_compute(ci + 1, 1)
            return 0

        lax.fori_loop(0, nchunks // 2, pair_body, 0)

    return c2(pts_flat, idx_all, gidx, alpha, beta)


def kernel(xyz, points, affine_alpha, affine_beta):
    B, N, C = points.shape
    S = N // _REDUCE
    fps_idx, new_xyz_t = _run_fps(xyz, S)
    fps_gidx = fps_idx + (jnp.arange(B, dtype=jnp.int32) * N)[:, None]
    pts_flat = points.reshape(B * N, C)
    idx4 = _run_ballq(new_xyz_t, xyz)
    idx_all = idx4.transpose(0, 1, 3, 2).reshape(B * S * _K)
    outp = _run_c2(pts_flat, idx_all, fps_gidx.reshape(B * S),
                   affine_alpha.reshape(C), affine_beta.reshape(C))
    new_xyz = new_xyz_t.transpose(0, 2, 1)
    new_points_out = outp.reshape(B, S, C).transpose(0, 2, 1)
    return (new_xyz, new_points_out)
